# baseline (speedup 1.0000x reference)
import functools

import jax
import jax.numpy as jnp
from jax import lax
from jax.experimental import pallas as pl
from jax.experimental.pallas import tpu as pltpu
from jax.experimental.pallas import tpu_sc as plsc

EMB = 64
NC = 2
NS = 16
NW = NC * NS

CHUNK = 64


def _sc_gather(ids_left, ids_right, table, n_pairs):
    p_per_w = n_pairs // NW
    n_chunks = p_per_w // CHUNK
    mesh = plsc.VectorSubcoreMesh(core_axis_name="c", subcore_axis_name="s")

    @functools.partial(
        pl.kernel,
        out_type=jax.ShapeDtypeStruct((n_pairs, 2 * EMB), jnp.bfloat16),
        mesh=mesh,
        scratch_types=[
            pltpu.VMEM((p_per_w,), jnp.int32),
            pltpu.VMEM((p_per_w,), jnp.int32),
            pltpu.VMEM((CHUNK, EMB), jnp.bfloat16),
            pltpu.VMEM((CHUNK, EMB), jnp.bfloat16),
            pltpu.SemaphoreType.DMA,
            pltpu.SemaphoreType.DMA,
        ],
        compiler_params=pltpu.CompilerParams(use_tc_tiling_on_sc=False),
    )
    def k(ids_l_hbm, ids_r_hbm, table_hbm, out_hbm, idx_l, idx_r, buf_l,
          buf_r, sem_l, sem_r):
        wid = lax.axis_index("s") * NC + lax.axis_index("c")
        base = wid * p_per_w
        pltpu.sync_copy(ids_l_hbm.at[pl.ds(base, p_per_w)], idx_l)
        pltpu.sync_copy(ids_r_hbm.at[pl.ds(base, p_per_w)], idx_r)

        def body(g, carry):
            off = g * CHUNK
            cl = pltpu.async_copy(
                table_hbm.at[idx_l.at[pl.ds(off, CHUNK)]], buf_l, sem_l,
            )
            cr = pltpu.async_copy(
                table_hbm.at[idx_r.at[pl.ds(off, CHUNK)]], buf_r, sem_r,
            )
            cl.wait()
            cr.wait()
            rows = out_hbm.at[pl.ds(base + off, CHUNK)]
            pltpu.sync_copy(buf_l, rows.at[:, pl.ds(0, EMB)])
            pltpu.sync_copy(buf_r, rows.at[:, pl.ds(EMB, EMB)])
            return carry

        lax.fori_loop(0, n_chunks, body, 0)

    return k(ids_left, ids_right, table)


def _tc_dense(x2, pos2, W2, H, b2bc, g2bc, be2bc, S, Bn):
    N2, L = x2.shape
    BH = N2 // S

    def body(x_ref, p_ref, w_ref, h_ref, b_ref, g_ref, be_ref, o_ref):
        i = pl.program_id(0)
        x = x_ref[...].astype(jnp.float32)
        xp = x + p_ref[pl.ds(i, 1), :]
        yt = lax.dot_general(
            w_ref[...], xp, (((1,), (1,)), ((), ())),
            preferred_element_type=jnp.float32,
            precision=lax.Precision.HIGHEST,
        ) + b_ref[...]
        mu = lax.dot_general(
            h_ref[...], yt, (((1,), (0,)), ((), ())),
            preferred_element_type=jnp.float32,
            precision=lax.Precision.DEFAULT,
        )
        ysq = lax.dot_general(
            h_ref[...], yt * yt, (((1,), (0,)), ((), ())),
            preferred_element_type=jnp.float32,
            precision=lax.Precision.DEFAULT,
        )
        var = ysq - mu * mu
        ot = (yt - mu) * lax.rsqrt(var + 1e-5) * g_ref[...] + be_ref[...]
        o_ref[...] = jnp.concatenate(
            [ot[0:EMB, :], ot[EMB:2 * EMB, :]], axis=1
        )[None]

    return pl.pallas_call(
        body,
        grid=(S,),
        in_specs=[
            pl.BlockSpec((BH, L), lambda i: (i, 0)),
            pl.BlockSpec((S, L), lambda i: (0, 0)),
            pl.BlockSpec((L, L), lambda i: (0, 0)),
            pl.BlockSpec((L, L), lambda i: (0, 0)),
            pl.BlockSpec((L, BH), lambda i: (0, 0)),
            pl.BlockSpec((L, BH), lambda i: (0, 0)),
            pl.BlockSpec((L, BH), lambda i: (0, 0)),
        ],
        out_specs=pl.BlockSpec((1, EMB, Bn), lambda i: (i, 0, 0)),
        out_shape=jax.ShapeDtypeStruct((S, EMB, Bn), jnp.float32),
    )(x2, pos2, W2, H, b2bc, g2bc, be2bc)


def kernel(input_ids, tok_table, pos_table, W, b, gamma, beta):
    Bn, S = input_ids.shape
    n_pairs = Bn * S // 2
    bh = Bn // 2

    ids_t = input_ids.T.astype(jnp.int32)
    ids_left = ids_t[:, :bh].reshape(-1)
    ids_right = ids_t[:, bh:].reshape(-1)

    x2 = _sc_gather(ids_left, ids_right,
                    tok_table.astype(jnp.bfloat16), n_pairs)

    Z = jnp.zeros((EMB, EMB), dtype=jnp.float32)
    W2 = jnp.block([[W, Z], [Z, W]])
    H = jnp.kron(jnp.eye(2, dtype=jnp.float32),
                 jnp.full((EMB, EMB), 1.0 / EMB, dtype=jnp.float32))
    pos2 = jnp.concatenate([pos_table, pos_table], axis=1)
    b2bc = jnp.broadcast_to(
        jnp.concatenate([b, b]).reshape(2 * EMB, 1), (2 * EMB, bh))
    g2bc = jnp.broadcast_to(
        jnp.concatenate([gamma, gamma]).reshape(2 * EMB, 1), (2 * EMB, bh))
    be2bc = jnp.broadcast_to(
        jnp.concatenate([beta, beta]).reshape(2 * EMB, 1), (2 * EMB, bh))

    out_t = _tc_dense(x2, pos2, W2, H, b2bc, g2bc, be2bc, S, Bn)
    return jnp.transpose(out_t, (2, 0, 1))


# speedup vs baseline: 1.5989x; 1.5989x over previous
import functools

import jax
import jax.numpy as jnp
from jax import lax
from jax.experimental import pallas as pl
from jax.experimental.pallas import tpu as pltpu
from jax.experimental.pallas import tpu_sc as plsc

EMB = 64
NC = 2
NS = 16
NW = NC * NS

CHUNK = 64


def _sc_gather(ids_left, ids_right, table, n_pairs):
    p_per_w = n_pairs // NW
    n_chunks = p_per_w // CHUNK
    mesh = plsc.VectorSubcoreMesh(core_axis_name="c", subcore_axis_name="s")

    @functools.partial(
        pl.kernel,
        out_type=jax.ShapeDtypeStruct((n_pairs, 2 * EMB), jnp.float32),
        mesh=mesh,
        scratch_types=[
            pltpu.VMEM((p_per_w,), jnp.int32),
            pltpu.VMEM((p_per_w,), jnp.int32),
            pltpu.VMEM((2, CHUNK, EMB), jnp.float32),
            pltpu.VMEM((2, CHUNK, EMB), jnp.float32),
            pltpu.SemaphoreType.DMA,
            pltpu.SemaphoreType.DMA,
        ],
        compiler_params=pltpu.CompilerParams(use_tc_tiling_on_sc=False),
    )
    def k(ids_l_hbm, ids_r_hbm, table_hbm, out_hbm, idx_l, idx_r, buf_l,
          buf_r, sem_l, sem_r):
        wid = lax.axis_index("s") * NC + lax.axis_index("c")
        base = wid * p_per_w
        pltpu.sync_copy(ids_l_hbm.at[pl.ds(base, p_per_w)], idx_l)
        pltpu.sync_copy(ids_r_hbm.at[pl.ds(base, p_per_w)], idx_r)

        def start(g, slot):
            off = g * CHUNK
            pltpu.async_copy(
                table_hbm.at[idx_l.at[pl.ds(off, CHUNK)]], buf_l.at[slot],
                sem_l,
            )
            pltpu.async_copy(
                table_hbm.at[idx_r.at[pl.ds(off, CHUNK)]], buf_r.at[slot],
                sem_r,
            )

        def finish(g, slot):
            off = g * CHUNK
            pltpu.make_async_copy(
                table_hbm.at[idx_l.at[pl.ds(off, CHUNK)]], buf_l.at[slot],
                sem_l,
            ).wait()
            pltpu.make_async_copy(
                table_hbm.at[idx_r.at[pl.ds(off, CHUNK)]], buf_r.at[slot],
                sem_r,
            ).wait()
            rows = out_hbm.at[pl.ds(base + off, CHUNK)]
            pltpu.sync_copy(buf_l.at[slot], rows.at[:, pl.ds(0, EMB)])
            pltpu.sync_copy(buf_r.at[slot], rows.at[:, pl.ds(EMB, EMB)])

        start(0, 0)

        def body(o, carry):
            g0 = o * 2
            start(g0 + 1, 1)
            finish(g0, 0)

            @pl.when(o < n_chunks // 2 - 1)
            def _():
                start(g0 + 2, 0)

            finish(g0 + 1, 1)
            return carry

        lax.fori_loop(0, n_chunks // 2, body, 0)

    return k(ids_left, ids_right, table)


def _tc_dense(x2, pos2, W2, H, b2bc, g2bc, be2bc, S, Bn):
    N2, L = x2.shape
    BH = N2 // S

    def body(x_ref, p_ref, w_ref, h_ref, b_ref, g_ref, be_ref, o_ref):
        i = pl.program_id(0)
        x = x_ref[...]
        xp = x + p_ref[pl.ds(i, 1), :]
        yt = lax.dot_general(
            w_ref[...], xp, (((1,), (1,)), ((), ())),
            preferred_element_type=jnp.float32,
            precision=lax.Precision.HIGHEST,
        ) + b_ref[...]
        mu = lax.dot_general(
            h_ref[...], yt, (((1,), (0,)), ((), ())),
            preferred_element_type=jnp.float32,
            precision=lax.Precision.DEFAULT,
        )
        ysq = lax.dot_general(
            h_ref[...], yt * yt, (((1,), (0,)), ((), ())),
            preferred_element_type=jnp.float32,
            precision=lax.Precision.DEFAULT,
        )
        var = ysq - mu * mu
        ot = (yt - mu) * lax.rsqrt(var + 1e-5) * g_ref[...] + be_ref[...]
        o_ref[...] = jnp.concatenate(
            [ot[0:EMB, :], ot[EMB:2 * EMB, :]], axis=1
        )[None]

    return pl.pallas_call(
        body,
        grid=(S,),
        in_specs=[
            pl.BlockSpec((BH, L), lambda i: (i, 0)),
            pl.BlockSpec((S, L), lambda i: (0, 0)),
            pl.BlockSpec((L, L), lambda i: (0, 0)),
            pl.BlockSpec((L, L), lambda i: (0, 0)),
            pl.BlockSpec((L, BH), lambda i: (0, 0)),
            pl.BlockSpec((L, BH), lambda i: (0, 0)),
            pl.BlockSpec((L, BH), lambda i: (0, 0)),
        ],
        out_specs=pl.BlockSpec((1, EMB, Bn), lambda i: (i, 0, 0)),
        out_shape=jax.ShapeDtypeStruct((S, EMB, Bn), jnp.float32),
    )(x2, pos2, W2, H, b2bc, g2bc, be2bc)


def kernel(input_ids, tok_table, pos_table, W, b, gamma, beta):
    Bn, S = input_ids.shape
    n_pairs = Bn * S // 2
    bh = Bn // 2

    ids_t = input_ids.T.astype(jnp.int32)
    ids_left = ids_t[:, :bh].reshape(-1)
    ids_right = ids_t[:, bh:].reshape(-1)

    x2 = _sc_gather(ids_left, ids_right, tok_table, n_pairs)

    Z = jnp.zeros((EMB, EMB), dtype=jnp.float32)
    W2 = jnp.block([[W, Z], [Z, W]])
    H = jnp.kron(jnp.eye(2, dtype=jnp.float32),
                 jnp.full((EMB, EMB), 1.0 / EMB, dtype=jnp.float32))
    pos2 = jnp.concatenate([pos_table, pos_table], axis=1)
    b2bc = jnp.broadcast_to(
        jnp.concatenate([b, b]).reshape(2 * EMB, 1), (2 * EMB, bh))
    g2bc = jnp.broadcast_to(
        jnp.concatenate([gamma, gamma]).reshape(2 * EMB, 1), (2 * EMB, bh))
    be2bc = jnp.broadcast_to(
        jnp.concatenate([beta, beta]).reshape(2 * EMB, 1), (2 * EMB, bh))

    out_t = _tc_dense(x2, pos2, W2, H, b2bc, g2bc, be2bc, S, Bn)
    return jnp.transpose(out_t, (2, 0, 1))


# speedup vs baseline: 1.7415x; 1.0892x over previous
import functools

import jax
import jax.numpy as jnp
from jax import lax
from jax.experimental import pallas as pl
from jax.experimental.pallas import tpu as pltpu
from jax.experimental.pallas import tpu_sc as plsc

EMB = 64
NC = 2
NS = 16
NW = NC * NS

CHUNK = 128


def _sc_gather(ids_left, ids_right, table, n_pairs):
    p_per_w = n_pairs // NW
    n_chunks = p_per_w // CHUNK
    mesh = plsc.VectorSubcoreMesh(core_axis_name="c", subcore_axis_name="s")

    @functools.partial(
        pl.kernel,
        out_type=jax.ShapeDtypeStruct((n_pairs, 2 * EMB), jnp.float32),
        mesh=mesh,
        scratch_types=[
            pltpu.VMEM((p_per_w,), jnp.int32),
            pltpu.VMEM((p_per_w,), jnp.int32),
            pltpu.VMEM((2, CHUNK, EMB), jnp.float32),
            pltpu.VMEM((2, CHUNK, EMB), jnp.float32),
            pltpu.SemaphoreType.DMA,
            pltpu.SemaphoreType.DMA,
        ],
        compiler_params=pltpu.CompilerParams(use_tc_tiling_on_sc=False),
    )
    def k(ids_l_hbm, ids_r_hbm, table_hbm, out_hbm, idx_l, idx_r, buf_l,
          buf_r, sem_l, sem_r):
        wid = lax.axis_index("s") * NC + lax.axis_index("c")
        base = wid * p_per_w
        pltpu.sync_copy(ids_l_hbm.at[pl.ds(base, p_per_w)], idx_l)
        pltpu.sync_copy(ids_r_hbm.at[pl.ds(base, p_per_w)], idx_r)

        def start(g, slot):
            off = g * CHUNK
            pltpu.async_copy(
                table_hbm.at[idx_l.at[pl.ds(off, CHUNK)]], buf_l.at[slot],
                sem_l,
            )
            pltpu.async_copy(
                table_hbm.at[idx_r.at[pl.ds(off, CHUNK)]], buf_r.at[slot],
                sem_r,
            )

        def finish(g, slot):
            off = g * CHUNK
            pltpu.make_async_copy(
                table_hbm.at[idx_l.at[pl.ds(off, CHUNK)]], buf_l.at[slot],
                sem_l,
            ).wait()
            pltpu.make_async_copy(
                table_hbm.at[idx_r.at[pl.ds(off, CHUNK)]], buf_r.at[slot],
                sem_r,
            ).wait()
            rows = out_hbm.at[pl.ds(base + off, CHUNK)]
            pltpu.sync_copy(buf_l.at[slot], rows.at[:, pl.ds(0, EMB)])
            pltpu.sync_copy(buf_r.at[slot], rows.at[:, pl.ds(EMB, EMB)])

        start(0, 0)

        def body(o, carry):
            g0 = o * 2
            start(g0 + 1, 1)
            finish(g0, 0)

            @pl.when(o < n_chunks // 2 - 1)
            def _():
                start(g0 + 2, 0)

            finish(g0 + 1, 1)
            return carry

        lax.fori_loop(0, n_chunks // 2, body, 0)

    return k(ids_left, ids_right, table)


def _tc_dense(x2, pos2, W2, H, b2bc, g2bc, be2bc, S, Bn):
    N2, L = x2.shape
    BH = N2 // S

    def body(x_ref, p_ref, w_ref, h_ref, b_ref, g_ref, be_ref, o_ref):
        i = pl.program_id(0)
        x = x_ref[...]
        xp = x + p_ref[pl.ds(i, 1), :]
        yt = lax.dot_general(
            w_ref[...], xp, (((1,), (1,)), ((), ())),
            preferred_element_type=jnp.float32,
            precision=lax.Precision.DEFAULT,
        ) + b_ref[...]
        mu = lax.dot_general(
            h_ref[...], yt, (((1,), (0,)), ((), ())),
            preferred_element_type=jnp.float32,
            precision=lax.Precision.DEFAULT,
        )
        ysq = lax.dot_general(
            h_ref[...], yt * yt, (((1,), (0,)), ((), ())),
            preferred_element_type=jnp.float32,
            precision=lax.Precision.DEFAULT,
        )
        var = ysq - mu * mu
        ot = (yt - mu) * lax.rsqrt(var + 1e-5) * g_ref[...] + be_ref[...]
        o_ref[...] = jnp.concatenate(
            [ot[0:EMB, :], ot[EMB:2 * EMB, :]], axis=1
        )[None]

    return pl.pallas_call(
        body,
        grid=(S,),
        in_specs=[
            pl.BlockSpec((BH, L), lambda i: (i, 0)),
            pl.BlockSpec((S, L), lambda i: (0, 0)),
            pl.BlockSpec((L, L), lambda i: (0, 0)),
            pl.BlockSpec((L, L), lambda i: (0, 0)),
            pl.BlockSpec((L, BH), lambda i: (0, 0)),
            pl.BlockSpec((L, BH), lambda i: (0, 0)),
            pl.BlockSpec((L, BH), lambda i: (0, 0)),
        ],
        out_specs=pl.BlockSpec((1, EMB, Bn), lambda i: (i, 0, 0)),
        out_shape=jax.ShapeDtypeStruct((S, EMB, Bn), jnp.float32),
    )(x2, pos2, W2, H, b2bc, g2bc, be2bc)


def kernel(input_ids, tok_table, pos_table, W, b, gamma, beta):
    Bn, S = input_ids.shape
    n_pairs = Bn * S // 2
    bh = Bn // 2

    ids_t = input_ids.T.astype(jnp.int32)
    ids_left = ids_t[:, :bh].reshape(-1)
    ids_right = ids_t[:, bh:].reshape(-1)

    x2 = _sc_gather(ids_left, ids_right, tok_table, n_pairs)

    Z = jnp.zeros((EMB, EMB), dtype=jnp.float32)
    W2 = jnp.block([[W, Z], [Z, W]])
    H = jnp.kron(jnp.eye(2, dtype=jnp.float32),
                 jnp.full((EMB, EMB), 1.0 / EMB, dtype=jnp.float32))
    pos2 = jnp.concatenate([pos_table, pos_table], axis=1)
    b2bc = jnp.broadcast_to(
        jnp.concatenate([b, b]).reshape(2 * EMB, 1), (2 * EMB, bh))
    g2bc = jnp.broadcast_to(
        jnp.concatenate([gamma, gamma]).reshape(2 * EMB, 1), (2 * EMB, bh))
    be2bc = jnp.broadcast_to(
        jnp.concatenate([beta, beta]).reshape(2 * EMB, 1), (2 * EMB, bh))

    out_t = _tc_dense(x2, pos2, W2, H, b2bc, g2bc, be2bc, S, Bn)
    return jnp.transpose(out_t, (2, 0, 1))
